# Initial kernel scaffold; baseline (speedup 1.0000x reference)
import jax
import jax.numpy as jnp
from jax.experimental import pallas as pl


def kernel(ques_emb, pad_answer, transform_matrix):
    raise NotImplementedError("write your pallas kernel here")


# speedup vs baseline: 5.0611x; 5.0611x over previous
import jax
import jax.numpy as jnp
from jax.experimental import pallas as pl

EMB = 128
ROWS_PER_BLOCK = 2048


def _tc_body(pad_ref, ques_ref, tm_ref, out_ref):
    q = ques_ref[...]
    m = pad_ref[...] == 0
    a_first = jnp.where(m, tm_ref[0:1, :EMB], tm_ref[1:2, :EMB])
    a_second = jnp.where(m, tm_ref[0:1, EMB:], tm_ref[1:2, EMB:])
    out_ref[:, :EMB] = q * a_first
    out_ref[:, EMB:] = q * a_second


def kernel(ques_emb, pad_answer, transform_matrix):
    B, L, D = ques_emb.shape
    N = B * L
    R = ROWS_PER_BLOCK
    assert N % R == 0
    ques2d = ques_emb.reshape(N, D)
    pad2d = pad_answer.reshape(N, 1).astype(jnp.int32)

    out = pl.pallas_call(
        _tc_body,
        grid=(N // R,),
        in_specs=[
            pl.BlockSpec((R, 1), lambda i: (i, 0)),
            pl.BlockSpec((R, D), lambda i: (i, 0)),
            pl.BlockSpec((2, 2 * D), lambda i: (0, 0)),
        ],
        out_specs=pl.BlockSpec((R, 2 * D), lambda i: (i, 0)),
        out_shape=jax.ShapeDtypeStruct((N, 2 * D), jnp.float32),
    )(pad2d, ques2d, transform_matrix)
    return out.reshape(B, L, 2 * D)


# speedup vs baseline: 5.5728x; 1.1011x over previous
import jax
import jax.numpy as jnp
from jax.experimental import pallas as pl

EMB = 128
ROWS_PER_BLOCK = 8192


def _tc_body(pad_ref, ques_ref, tm_ref, out_ref):
    q = ques_ref[...]
    m = pad_ref[...] == 0
    a_first = jnp.where(m, tm_ref[0:1, :EMB], tm_ref[1:2, :EMB])
    a_second = jnp.where(m, tm_ref[0:1, EMB:], tm_ref[1:2, EMB:])
    out_ref[:, :EMB] = q * a_first
    out_ref[:, EMB:] = q * a_second


def kernel(ques_emb, pad_answer, transform_matrix):
    B, L, D = ques_emb.shape
    N = B * L
    R = ROWS_PER_BLOCK
    assert N % R == 0
    ques2d = ques_emb.reshape(N, D)
    pad2d = pad_answer.reshape(N, 1).astype(jnp.int32)

    out = pl.pallas_call(
        _tc_body,
        grid=(N // R,),
        in_specs=[
            pl.BlockSpec((R, 1), lambda i: (i, 0)),
            pl.BlockSpec((R, D), lambda i: (i, 0)),
            pl.BlockSpec((2, 2 * D), lambda i: (0, 0)),
        ],
        out_specs=pl.BlockSpec((R, 2 * D), lambda i: (i, 0)),
        out_shape=jax.ShapeDtypeStruct((N, 2 * D), jnp.float32),
    )(pad2d, ques2d, transform_matrix)
    return out.reshape(B, L, 2 * D)


# speedup vs baseline: 9.4595x; 1.6974x over previous
import jax
import jax.numpy as jnp
from jax import lax
from jax.experimental import pallas as pl

EMB = 128
ROWS_PER_BLOCK = 8192


def _tc_body(pad_ref, ques_ref, tm_ref, out_ref):
    R = ques_ref.shape[0]
    G = R // 128
    padf = pad_ref[...].astype(jnp.float32)
    row = lax.broadcasted_iota(jnp.int32, (128, 128), 0)
    col = lax.broadcasted_iota(jnp.int32, (128, 128), 1)
    eye = jnp.where(row == col, 1.0, 0.0).astype(jnp.float32)
    padt = lax.dot_general(eye, padf, (((1,), (1,)), ((), ())),
                           preferred_element_type=jnp.float32)
    tm0a = tm_ref[0:1, :EMB]
    d1a = tm_ref[1:2, :EMB] - tm0a
    tm0b = tm_ref[0:1, EMB:]
    d1b = tm_ref[1:2, EMB:] - tm0b
    for g in range(G):
        c = padt[:, g:g + 1]
        q = ques_ref[pl.ds(g * 128, 128), :]
        out_ref[pl.ds(g * 128, 128), :EMB] = q * (tm0a + c * d1a)
        out_ref[pl.ds(g * 128, 128), EMB:] = q * (tm0b + c * d1b)


def kernel(ques_emb, pad_answer, transform_matrix):
    B, L, D = ques_emb.shape
    N = B * L
    R = ROWS_PER_BLOCK
    assert N % R == 0 and R % 128 == 0
    ques2d = ques_emb.reshape(N, D)
    pad2d = pad_answer.reshape(N // 128, 128).astype(jnp.int32)

    out = pl.pallas_call(
        _tc_body,
        grid=(N // R,),
        in_specs=[
            pl.BlockSpec((R // 128, 128), lambda i: (i, 0)),
            pl.BlockSpec((R, D), lambda i: (i, 0)),
            pl.BlockSpec((2, 2 * D), lambda i: (0, 0)),
        ],
        out_specs=pl.BlockSpec((R, 2 * D), lambda i: (i, 0)),
        out_shape=jax.ShapeDtypeStruct((N, 2 * D), jnp.float32),
    )(pad2d, ques2d, transform_matrix)
    return out.reshape(B, L, 2 * D)
